# baseline (speedup 1.0000x reference)
import functools

import jax
import jax.numpy as jnp
from jax import lax
from jax.experimental import pallas as pl
from jax.experimental.pallas import tpu as pltpu
from jax.experimental.pallas import tpu_sc as plsc

BATCH = 16384
VOCAB = 1000000
D_FEN = 64
NUM_CORES = 2
NUM_SUBCORES = 16
NUM_WORKERS = NUM_CORES * NUM_SUBCORES
B_PER_W = BATCH // NUM_WORKERS
IDX_CHUNK = 128
N_CHUNKS = B_PER_W // IDX_CHUNK
LANES = 16
N_GROUPS = B_PER_W // LANES

ABLK = 65536
NA = (VOCAB + ABLK - 1) // ABLK


def _planes_body(t_ref, w_ref, st_ref, tt_ref, b_ref,
                 l0_ref, l1_ref, l2_ref, sk_ref, tm_ref):
    f32 = jnp.float32
    blk = t_ref[...]
    wf = w_ref[:, 0:64]
    out = lax.dot_general(wf, blk, (((1,), (0,)), ((), ())),
                          preferred_element_type=f32)
    l0_ref[...] = out[0]
    l1_ref[...] = out[1]
    l2_ref[...] = out[2]
    zeros = jnp.zeros((3, 8), f32)
    sk = lax.dot_general(
        w_ref[:, 64:96], st_ref[...], (((1,), (1,)), ((), ())),
        preferred_element_type=f32) + b_ref[...].reshape(3, 1)
    tm = lax.dot_general(
        w_ref[:, 96:128], tt_ref[...], (((1,), (1,)), ((), ())),
        preferred_element_type=f32)
    sk_ref[...] = jnp.concatenate([sk, zeros], axis=1)
    tm_ref[...] = jnp.concatenate([tm, zeros, zeros[:, 0:3]], axis=1)


def _planes(table_t, W_t, skill_table, time_table, b):
    return pl.pallas_call(
        _planes_body,
        grid=(NA,),
        in_specs=[
            pl.BlockSpec((D_FEN, ABLK), lambda i: (0, i)),
            pl.BlockSpec((3, 128), lambda i: (0, 0)),
            pl.BlockSpec((8, 32), lambda i: (0, 0)),
            pl.BlockSpec((5, 32), lambda i: (0, 0)),
            pl.BlockSpec((3,), lambda i: (0,)),
        ],
        out_specs=[
            pl.BlockSpec((ABLK,), lambda i: (i,)),
            pl.BlockSpec((ABLK,), lambda i: (i,)),
            pl.BlockSpec((ABLK,), lambda i: (i,)),
            pl.BlockSpec((3, 16), lambda i: (0, 0)),
            pl.BlockSpec((3, 16), lambda i: (0, 0)),
        ],
        out_shape=[jax.ShapeDtypeStruct((VOCAB,), jnp.float32)] * 3
        + [jax.ShapeDtypeStruct((3, 16), jnp.float32),
           jax.ShapeDtypeStruct((3, 16), jnp.float32)],
    )(table_t, W_t, skill_table, time_table, b)


def _vgather(v, idx):
    return lax.gather(
        v, idx[:, None],
        dimension_numbers=lax.GatherDimensionNumbers(
            offset_dims=(), collapsed_slice_dims=(0,), start_index_map=(0,)),
        slice_sizes=(1,),
        mode=lax.GatherScatterMode.PROMISE_IN_BOUNDS)


@functools.cache
def _make_gather_head():
    mesh = plsc.VectorSubcoreMesh(core_axis_name="c", subcore_axis_name="s")

    @functools.partial(
        pl.kernel,
        mesh=mesh,
        out_type=jax.ShapeDtypeStruct((3, BATCH), jnp.float32),
        scratch_types=[
            pltpu.VMEM((N_CHUNKS, IDX_CHUNK), jnp.int32),
            pltpu.VMEM((3, B_PER_W), jnp.float32),
            pltpu.VMEM((B_PER_W,), jnp.int32),
            pltpu.VMEM((B_PER_W,), jnp.int32),
            pltpu.VMEM((3, 16), jnp.float32),
            pltpu.VMEM((3, 16), jnp.float32),
            pltpu.SemaphoreType.DMA,
        ],
        compiler_params=pltpu.CompilerParams(use_tc_tiling_on_sc=False),
    )
    def _gather_head(idx_hbm, l0_hbm, l1_hbm, l2_hbm, sk_hbm, tm_hbm,
                     sidx_hbm, tidx_hbm, out_hbm,
                     idx_v, g_v, sk_v, tm_v, sklog_v, tmlog_v, sem):
        i32, f32 = jnp.int32, jnp.float32
        wid = lax.axis_index("s") * NUM_CORES + lax.axis_index("c")
        base = wid * B_PER_W
        pltpu.sync_copy(idx_hbm.at[wid], idx_v)
        pltpu.sync_copy(sidx_hbm.at[wid], sk_v)
        pltpu.sync_copy(tidx_hbm.at[wid], tm_v)
        pltpu.sync_copy(sk_hbm, sklog_v)
        pltpu.sync_copy(tm_hbm, tmlog_v)
        copies = []
        for p, plane in enumerate((l0_hbm, l1_hbm, l2_hbm)):
            for c in range(N_CHUNKS):
                copies.append(
                    pltpu.async_copy(
                        plane.at[idx_v.at[c]],
                        g_v.at[p, pl.ds(c * IDX_CHUNK, IDX_CHUNK)],
                        sem,
                    )
                )
        for cp in copies:
            cp.wait()
        sk_vec = [sklog_v[c] for c in range(3)]
        tm_vec = [tmlog_v[c] for c in range(3)]
        for grp in range(N_GROUPS):
            sl = pl.ds(grp * LANES, LANES)
            sidx = sk_v[sl]
            tidx = tm_v[sl]
            lg = []
            for c in range(3):
                skc = _vgather(sk_vec[c], sidx)
                tmc = _vgather(tm_vec[c], tidx)
                lg.append(g_v[c, sl] + skc + tmc)
            m = jnp.maximum(jnp.maximum(lg[0], lg[1]), lg[2])
            e = [jnp.exp(x - m) for x in lg]
            s = e[0] + e[1] + e[2]
            p3 = [x / s for x in e]
            p3 = [jnp.clip(x, 0.0, 1.0) for x in p3]
            t = p3[0] + p3[1] + p3[2]
            for c in range(3):
                g_v[c, sl] = jnp.where(t > 0.0, p3[c] / t, p3[c])
        pltpu.sync_copy(g_v, out_hbm.at[:, pl.ds(base, B_PER_W)])

    return _gather_head


def kernel(fen_idx, skill_idx, time_idx, fen_table, skill_table, time_table, W, b):
    fen_idx = fen_idx.astype(jnp.int32).reshape(NUM_WORKERS, N_CHUNKS, IDX_CHUNK)
    skill_idx = skill_idx.reshape(NUM_WORKERS, B_PER_W)
    time_idx = time_idx.reshape(NUM_WORKERS, B_PER_W)
    table_t = jnp.transpose(fen_table)
    W_t = jnp.transpose(W)
    l0, l1, l2, sk_log, tm_log = _planes(table_t, W_t, skill_table, time_table, b)
    probs_t = _make_gather_head()(fen_idx, l0, l1, l2, sk_log, tm_log,
                                  skill_idx, time_idx)
    return jnp.transpose(probs_t)


# speedup vs baseline: 1.0487x; 1.0487x over previous
import functools

import jax
import jax.numpy as jnp
from jax import lax
from jax.experimental import pallas as pl
from jax.experimental.pallas import tpu as pltpu
from jax.experimental.pallas import tpu_sc as plsc

BATCH = 16384
VOCAB = 1000000
D_FEN = 64
NUM_CORES = 2
NUM_SUBCORES = 16
NUM_WORKERS = NUM_CORES * NUM_SUBCORES
B_PER_W = BATCH // NUM_WORKERS
IDX_CHUNK = 128
N_CHUNKS = B_PER_W // IDX_CHUNK

ABLK = 98304
NA = (VOCAB + ABLK - 1) // ABLK


def _planes_body(t_ref, w_ref, l0_ref, l1_ref, l2_ref):
    blk = t_ref[...]
    wf = w_ref[:, 0:64]
    out = lax.dot_general(wf, blk, (((1,), (0,)), ((), ())),
                          preferred_element_type=jnp.float32)
    l0_ref[...] = out[0]
    l1_ref[...] = out[1]
    l2_ref[...] = out[2]


def _planes(table_t, W_t):
    return pl.pallas_call(
        _planes_body,
        grid=(NA,),
        in_specs=[
            pl.BlockSpec((D_FEN, ABLK), lambda i: (0, i)),
            pl.BlockSpec((3, 128), lambda i: (0, 0)),
        ],
        out_specs=[
            pl.BlockSpec((ABLK,), lambda i: (i,)),
            pl.BlockSpec((ABLK,), lambda i: (i,)),
            pl.BlockSpec((ABLK,), lambda i: (i,)),
        ],
        out_shape=[jax.ShapeDtypeStruct((VOCAB,), jnp.float32)] * 3,
    )(table_t, W_t)


@functools.cache
def _make_plane_gather():
    mesh = plsc.VectorSubcoreMesh(core_axis_name="c", subcore_axis_name="s")

    @functools.partial(
        pl.kernel,
        mesh=mesh,
        out_type=[jax.ShapeDtypeStruct((BATCH,), jnp.float32)] * 3,
        scratch_types=[
            pltpu.VMEM((N_CHUNKS, IDX_CHUNK), jnp.int32),
            pltpu.VMEM((3, B_PER_W), jnp.float32),
            pltpu.SemaphoreType.DMA,
        ],
        compiler_params=pltpu.CompilerParams(use_tc_tiling_on_sc=False),
    )
    def _plane_gather(idx_hbm, l0_hbm, l1_hbm, l2_hbm,
                      g0_hbm, g1_hbm, g2_hbm, idx_v, g_v, sem):
        wid = lax.axis_index("s") * NUM_CORES + lax.axis_index("c")
        base = wid * B_PER_W
        pltpu.sync_copy(idx_hbm.at[wid], idx_v)
        copies = []
        for p, plane in enumerate((l0_hbm, l1_hbm, l2_hbm)):
            for c in range(N_CHUNKS):
                copies.append(
                    pltpu.async_copy(
                        plane.at[idx_v.at[c]],
                        g_v.at[p, pl.ds(c * IDX_CHUNK, IDX_CHUNK)],
                        sem,
                    )
                )
        for cp in copies:
            cp.wait()
        for p, g_out in enumerate((g0_hbm, g1_hbm, g2_hbm)):
            pltpu.sync_copy(g_v.at[p], g_out.at[pl.ds(base, B_PER_W)])

    return _plane_gather


def _head_body(g0_ref, g1_ref, g2_ref, sidx_ref, tidx_ref, st_ref, tt_ref,
               w_ref, b_ref, out_ref):
    f32 = jnp.float32
    logits = jnp.concatenate(
        [g0_ref[...].reshape(1, BATCH), g1_ref[...].reshape(1, BATCH),
         g2_ref[...].reshape(1, BATCH)], axis=0)
    sk_log = lax.dot_general(w_ref[:, 64:96], st_ref[...], (((1,), (1,)), ((), ())),
                             preferred_element_type=f32)
    tm_log = lax.dot_general(w_ref[:, 96:128], tt_ref[...], (((1,), (1,)), ((), ())),
                             preferred_element_type=f32)
    sidx = sidx_ref[...].reshape(1, BATCH)
    tidx = tidx_ref[...].reshape(1, BATCH)
    soh = (lax.broadcasted_iota(jnp.int32, (8, BATCH), 0) == sidx).astype(f32)
    toh = (lax.broadcasted_iota(jnp.int32, (5, BATCH), 0) == tidx).astype(f32)
    logits = logits + lax.dot_general(sk_log, soh, (((1,), (0,)), ((), ())),
                                      preferred_element_type=f32)
    logits = logits + lax.dot_general(tm_log, toh, (((1,), (0,)), ((), ())),
                                      preferred_element_type=f32)
    logits = logits + b_ref[...].reshape(3, 1)
    m = jnp.max(logits, axis=0, keepdims=True)
    e = jnp.exp(logits - m)
    s = jnp.sum(e, axis=0, keepdims=True)
    p = e / s
    p = jnp.clip(p, 0.0, 1.0)
    t = jnp.sum(p, axis=0, keepdims=True)
    out_ref[...] = jnp.where(t > 0.0, p / t, p)


def _head(g0, g1, g2, skill_idx, time_idx, skill_table, time_table, W_t, b):
    return pl.pallas_call(
        _head_body,
        in_specs=[
            pl.BlockSpec((BATCH,), lambda: (0,)),
            pl.BlockSpec((BATCH,), lambda: (0,)),
            pl.BlockSpec((BATCH,), lambda: (0,)),
            pl.BlockSpec((BATCH,), lambda: (0,)),
            pl.BlockSpec((BATCH,), lambda: (0,)),
            pl.BlockSpec((8, 32), lambda: (0, 0)),
            pl.BlockSpec((5, 32), lambda: (0, 0)),
            pl.BlockSpec((3, 128), lambda: (0, 0)),
            pl.BlockSpec((3,), lambda: (0,)),
        ],
        out_specs=pl.BlockSpec((3, BATCH), lambda: (0, 0)),
        out_shape=jax.ShapeDtypeStruct((3, BATCH), jnp.float32),
    )(g0, g1, g2, skill_idx, time_idx, skill_table, time_table, W_t, b)


def kernel(fen_idx, skill_idx, time_idx, fen_table, skill_table, time_table, W, b):
    fen_idx = fen_idx.astype(jnp.int32).reshape(NUM_WORKERS, N_CHUNKS, IDX_CHUNK)
    table_t = jnp.transpose(fen_table)
    W_t = jnp.transpose(W)
    l0, l1, l2 = _planes(table_t, W_t)
    g0, g1, g2 = _make_plane_gather()(fen_idx, l0, l1, l2)
    probs_t = _head(g0, g1, g2, skill_idx, time_idx, skill_table, time_table,
                    W_t, b)
    return jnp.transpose(probs_t)


# speedup vs baseline: 1.0541x; 1.0051x over previous
import functools

import jax
import jax.numpy as jnp
from jax import lax
from jax.experimental import pallas as pl
from jax.experimental.pallas import tpu as pltpu
from jax.experimental.pallas import tpu_sc as plsc

BATCH = 16384
VOCAB = 1000000
D_FEN = 64
NUM_CORES = 2
NUM_SUBCORES = 16
NUM_WORKERS = NUM_CORES * NUM_SUBCORES
B_PER_W = BATCH // NUM_WORKERS
IDX_CHUNK = 128
N_CHUNKS = B_PER_W // IDX_CHUNK

ABLK = 65536
NA = (VOCAB + ABLK - 1) // ABLK


def _planes_body(t_ref, w_ref, l0_ref, l1_ref, l2_ref):
    blk = t_ref[...]
    wf = w_ref[:, 0:64]
    out = lax.dot_general(wf, blk, (((1,), (0,)), ((), ())),
                          preferred_element_type=jnp.float32)
    l0_ref[...] = out[0]
    l1_ref[...] = out[1]
    l2_ref[...] = out[2]


def _planes(table_t, W_t):
    return pl.pallas_call(
        _planes_body,
        grid=(NA,),
        in_specs=[
            pl.BlockSpec((D_FEN, ABLK), lambda i: (0, i)),
            pl.BlockSpec((3, 128), lambda i: (0, 0)),
        ],
        out_specs=[
            pl.BlockSpec((ABLK,), lambda i: (i,)),
            pl.BlockSpec((ABLK,), lambda i: (i,)),
            pl.BlockSpec((ABLK,), lambda i: (i,)),
        ],
        out_shape=[jax.ShapeDtypeStruct((VOCAB,), jnp.float32)] * 3,
    )(table_t, W_t)


@functools.cache
def _make_plane_gather():
    mesh = plsc.VectorSubcoreMesh(core_axis_name="c", subcore_axis_name="s")

    @functools.partial(
        pl.kernel,
        mesh=mesh,
        out_type=[jax.ShapeDtypeStruct((BATCH,), jnp.float32)] * 3,
        scratch_types=[
            pltpu.VMEM((N_CHUNKS, IDX_CHUNK), jnp.int32),
            pltpu.VMEM((3, B_PER_W), jnp.float32),
            pltpu.SemaphoreType.DMA,
        ],
        compiler_params=pltpu.CompilerParams(use_tc_tiling_on_sc=False),
    )
    def _plane_gather(idx_hbm, l0_hbm, l1_hbm, l2_hbm,
                      g0_hbm, g1_hbm, g2_hbm, idx_v, g_v, sem):
        wid = lax.axis_index("s") * NUM_CORES + lax.axis_index("c")
        base = wid * B_PER_W
        pltpu.sync_copy(idx_hbm.at[wid], idx_v)
        copies = []
        for p, plane in enumerate((l0_hbm, l1_hbm, l2_hbm)):
            for c in range(N_CHUNKS):
                copies.append(
                    pltpu.async_copy(
                        plane.at[idx_v.at[c]],
                        g_v.at[p, pl.ds(c * IDX_CHUNK, IDX_CHUNK)],
                        sem,
                    )
                )
        for cp in copies:
            cp.wait()
        for p, g_out in enumerate((g0_hbm, g1_hbm, g2_hbm)):
            pltpu.sync_copy(g_v.at[p], g_out.at[pl.ds(base, B_PER_W)])

    return _plane_gather


def _head_body(g0_ref, g1_ref, g2_ref, sidx_ref, tidx_ref, st_ref, tt_ref,
               w_ref, b_ref, out_ref):
    f32 = jnp.float32
    logits = jnp.concatenate(
        [g0_ref[...].reshape(1, BATCH), g1_ref[...].reshape(1, BATCH),
         g2_ref[...].reshape(1, BATCH)], axis=0)
    sk_log = lax.dot_general(w_ref[:, 64:96], st_ref[...], (((1,), (1,)), ((), ())),
                             preferred_element_type=f32)
    tm_log = lax.dot_general(w_ref[:, 96:128], tt_ref[...], (((1,), (1,)), ((), ())),
                             preferred_element_type=f32)
    sidx = sidx_ref[...].reshape(1, BATCH)
    tidx = tidx_ref[...].reshape(1, BATCH)
    soh = (lax.broadcasted_iota(jnp.int32, (8, BATCH), 0) == sidx).astype(f32)
    toh = (lax.broadcasted_iota(jnp.int32, (5, BATCH), 0) == tidx).astype(f32)
    logits = logits + lax.dot_general(sk_log, soh, (((1,), (0,)), ((), ())),
                                      preferred_element_type=f32)
    logits = logits + lax.dot_general(tm_log, toh, (((1,), (0,)), ((), ())),
                                      preferred_element_type=f32)
    logits = logits + b_ref[...].reshape(3, 1)
    m = jnp.max(logits, axis=0, keepdims=True)
    e = jnp.exp(logits - m)
    s = jnp.sum(e, axis=0, keepdims=True)
    p = e / s
    p = jnp.clip(p, 0.0, 1.0)
    t = jnp.sum(p, axis=0, keepdims=True)
    out_ref[...] = jnp.where(t > 0.0, p / t, p)


def _head(g0, g1, g2, skill_idx, time_idx, skill_table, time_table, W_t, b):
    return pl.pallas_call(
        _head_body,
        in_specs=[
            pl.BlockSpec((BATCH,), lambda: (0,)),
            pl.BlockSpec((BATCH,), lambda: (0,)),
            pl.BlockSpec((BATCH,), lambda: (0,)),
            pl.BlockSpec((BATCH,), lambda: (0,)),
            pl.BlockSpec((BATCH,), lambda: (0,)),
            pl.BlockSpec((8, 32), lambda: (0, 0)),
            pl.BlockSpec((5, 32), lambda: (0, 0)),
            pl.BlockSpec((3, 128), lambda: (0, 0)),
            pl.BlockSpec((3,), lambda: (0,)),
        ],
        out_specs=pl.BlockSpec((3, BATCH), lambda: (0, 0)),
        out_shape=jax.ShapeDtypeStruct((3, BATCH), jnp.float32),
    )(g0, g1, g2, skill_idx, time_idx, skill_table, time_table, W_t, b)


def kernel(fen_idx, skill_idx, time_idx, fen_table, skill_table, time_table, W, b):
    fen_idx = fen_idx.astype(jnp.int32).reshape(NUM_WORKERS, N_CHUNKS, IDX_CHUNK)
    table_t = jnp.transpose(fen_table)
    W_t = jnp.transpose(W)
    l0, l1, l2 = _planes(table_t, W_t)
    g0, g1, g2 = _make_plane_gather()(fen_idx, l0, l1, l2)
    probs_t = _head(g0, g1, g2, skill_idx, time_idx, skill_table, time_table,
                    W_t, b)
    return jnp.transpose(probs_t)
